# baseline (speedup 1.0000x reference)
import functools

import jax
import jax.numpy as jnp
from jax import lax
from jax.experimental import pallas as pl
from jax.experimental.pallas import tpu as pltpu
from jax.experimental.pallas import tpu_sc as plsc

_DIM = 64
_N_EMBED = 1024
_BETA = 0.25

_TILE_M = 576
_PAD_M = 640


def _dist_body(xt_ref, e_ref, ind_ref, loss_ref, t3_ref):
    pid = pl.program_id(0)
    x = xt_ref[0].T
    e = e_ref[...]
    s = jax.lax.dot_general(
        x, e, (((1,), (0,)), ((), ())), preferred_element_type=jnp.float32
    )
    x2 = jnp.sum(x * x, axis=1, keepdims=True)
    e2 = jnp.sum(e * e, axis=0, keepdims=True)
    dist = x2 - 2.0 * s + e2
    m = jnp.min(dist, axis=1, keepdims=True)
    cols = jax.lax.broadcasted_iota(jnp.int32, dist.shape, 1).astype(jnp.float32)
    hit = jnp.where(dist <= m, cols, float(_N_EMBED))
    ind_ref[0, 0, :_TILE_M] = jnp.min(hit, axis=1).astype(jnp.int32)
    part = jnp.sum(m)

    @pl.when(pid == 0)
    def _():
        loss_ref[0, 0] = 0.0
        for k in range(_N_EMBED // 128):
            t3_ref[:, k, :] = e[:, 128 * k:128 * (k + 1)]

    loss_ref[0, 0] += part


def _dist_argmin(input_t, embed):
    count = input_t.shape[0]
    ind, loss_sum, table3 = pl.pallas_call(
        _dist_body,
        grid=(count,),
        in_specs=[
            pl.BlockSpec((1, _DIM, _TILE_M), lambda i: (i, 0, 0)),
            pl.BlockSpec((_DIM, _N_EMBED), lambda i: (0, 0)),
        ],
        out_specs=[
            pl.BlockSpec((1, 1, _PAD_M), lambda i: (i, 0, 0)),
            pl.BlockSpec(memory_space=pltpu.SMEM),
            pl.BlockSpec((_DIM, _N_EMBED // 128, 128), lambda i: (0, 0, 0)),
        ],
        out_shape=[
            jax.ShapeDtypeStruct((count, 1, _PAD_M), jnp.int32),
            jax.ShapeDtypeStruct((1, 1), jnp.float32),
            jax.ShapeDtypeStruct((_DIM, _N_EMBED // 128, 128), jnp.float32),
        ],
    )(input_t, embed)
    return ind, loss_sum[0, 0], table3


def _make_gather(nb):
    dh = _DIM * nb // 32
    groups = _TILE_M // 16
    mesh = plsc.VectorSubcoreMesh(core_axis_name="c", subcore_axis_name="s")

    @functools.partial(
        pl.kernel,
        mesh=mesh,
        compiler_params=pltpu.CompilerParams(
            use_tc_tiling_on_sc=False,
            needs_layout_passes=False,
            disable_bounds_checks=True,
        ),
        out_type=jax.ShapeDtypeStruct((nb, _DIM, _PAD_M), jnp.float32),
        scratch_types=[
            pltpu.VMEM((dh, _N_EMBED // 128, 128), jnp.float32),
            pltpu.VMEM((_PAD_M,), jnp.int32),
            pltpu.VMEM((dh, _PAD_M), jnp.float32),
            pltpu.SemaphoreType.DMA,
        ],
    )
    def gather(table_hbm, idx_hbm, out_hbm, table_v, idx_v, rows_v, sem):
        wid = lax.axis_index("s") * 2 + lax.axis_index("c")
        nsplit = 32 // nb
        b = wid // nsplit
        d0 = (wid % nsplit) * dh
        cp = pltpu.async_copy(table_hbm.at[pl.ds(d0, dh)], table_v, sem)
        pltpu.sync_copy(idx_hbm.at[b, 0], idx_v)
        cp.wait()
        lanes = lax.iota(jnp.int32, 16)
        zeros = lanes * 0

        def body(g, carry):
            idx16 = idx_v[pl.ds(g * 16, 16)]
            k16 = lax.shift_right_logical(idx16, 7)
            l16 = lax.bitwise_and(idx16, 127)
            d16 = zeros
            for d4 in range(dh // 4):
                vs = []
                for _ in range(4):
                    vs.append(plsc.load_gather(table_v, [d16, k16, l16]))
                    d16 = d16 + 1
                for j, v in enumerate(vs):
                    rows_v[d4 * 4 + j, pl.ds(g * 16, 16)] = v
            return carry

        lax.fori_loop(0, groups, body, 0)
        pltpu.sync_copy(rows_v, out_hbm.at[b, pl.ds(d0, dh)])

    return gather


def kernel(input, embed):
    b, t, c = input.shape
    h = b // 2
    input_t = jnp.swapaxes(input, 1, 2)
    gather = _make_gather(h)
    ind_a, loss_a, table3 = _dist_argmin(input_t[:h], embed)
    qa = gather(table3, ind_a)
    ind_b, loss_b, _ = _dist_argmin(input_t[h:], embed)
    qb = gather(table3, ind_b)
    qpad = jnp.concatenate([qa, qb], axis=0)
    quantize = jnp.swapaxes(qpad[:, :, :t], 1, 2)
    loss = (loss_a + loss_b) * (_BETA / (b * t * c))
    ind = jnp.concatenate([ind_a, ind_b], axis=0)[:, 0, :t]
    return quantize, loss, ind


# speedup vs baseline: 1.1199x; 1.1199x over previous
import functools

import jax
import jax.numpy as jnp
from jax import lax
from jax.experimental import pallas as pl
from jax.experimental.pallas import tpu as pltpu
from jax.experimental.pallas import tpu_sc as plsc

_DIM = 64
_N_EMBED = 1024
_BETA = 0.25

_TILE_M = 576
_PAD_M = 640


def _dist_body(xt_ref, e_ref, ind_ref, loss_ref, t3_ref):
    pid = pl.program_id(0)
    x = xt_ref[0].T
    e = e_ref[...]
    s = jax.lax.dot_general(
        x, e, (((1,), (0,)), ((), ())), preferred_element_type=jnp.float32
    )
    x2 = jnp.sum(x * x, axis=1, keepdims=True)
    e2 = jnp.sum(e * e, axis=0, keepdims=True)
    dist = x2 - 2.0 * s + e2
    m = jnp.min(dist, axis=1, keepdims=True)
    cols = jax.lax.broadcasted_iota(jnp.int32, dist.shape, 1).astype(jnp.float32)
    hit = jnp.where(dist <= m, cols, float(_N_EMBED))
    ind_ref[0, 0, :_TILE_M] = jnp.min(hit, axis=1).astype(jnp.int32)
    part = jnp.sum(m)

    @pl.when(pid == 0)
    def _():
        loss_ref[0, 0] = 0.0
        for k in range(_N_EMBED // 128):
            t3_ref[:, k, :] = e[:, 128 * k:128 * (k + 1)]

    loss_ref[0, 0] += part


def _dist_argmin(input_t, embed):
    count = input_t.shape[0]
    ind, loss_sum, table3 = pl.pallas_call(
        _dist_body,
        grid=(count,),
        in_specs=[
            pl.BlockSpec((1, _DIM, _TILE_M), lambda i: (i, 0, 0)),
            pl.BlockSpec((_DIM, _N_EMBED), lambda i: (0, 0)),
        ],
        out_specs=[
            pl.BlockSpec((1, 1, _PAD_M), lambda i: (i, 0, 0)),
            pl.BlockSpec(memory_space=pltpu.SMEM),
            pl.BlockSpec((_DIM, _N_EMBED // 128, 128), lambda i: (0, 0, 0)),
        ],
        out_shape=[
            jax.ShapeDtypeStruct((count, 1, _PAD_M), jnp.int32),
            jax.ShapeDtypeStruct((1, 1), jnp.float32),
            jax.ShapeDtypeStruct((_DIM, _N_EMBED // 128, 128), jnp.float32),
        ],
    )(input_t, embed)
    return ind, loss_sum[0, 0], table3


def _make_gather(nb):
    dh = _DIM * nb // 32
    groups = _TILE_M // 16
    mesh = plsc.VectorSubcoreMesh(core_axis_name="c", subcore_axis_name="s")

    @functools.partial(
        pl.kernel,
        mesh=mesh,
        compiler_params=pltpu.CompilerParams(
            use_tc_tiling_on_sc=False,
            needs_layout_passes=False,
            disable_bounds_checks=True,
        ),
        out_type=jax.ShapeDtypeStruct((nb, _DIM, _PAD_M), jnp.float32),
        scratch_types=[
            pltpu.VMEM((dh, _N_EMBED // 128, 128), jnp.float32),
            pltpu.VMEM((_PAD_M,), jnp.int32),
            pltpu.VMEM((dh, _PAD_M), jnp.float32),
            pltpu.SemaphoreType.DMA,
        ],
    )
    def gather(table_hbm, idx_hbm, out_hbm, table_v, idx_v, rows_v, sem):
        wid = lax.axis_index("s") * 2 + lax.axis_index("c")
        nsplit = 32 // nb
        b = wid // nsplit
        d0 = (wid % nsplit) * dh
        cp = pltpu.async_copy(table_hbm.at[pl.ds(d0, dh)], table_v, sem)
        pltpu.sync_copy(idx_hbm.at[b, 0], idx_v)
        cp.wait()
        lanes = lax.iota(jnp.int32, 16)
        zeros = lanes * 0

        def body(g, carry):
            idx16 = idx_v[pl.ds(g * 16, 16)]
            k16 = lax.shift_right_logical(idx16, 7)
            l16 = lax.bitwise_and(idx16, 127)
            d16 = zeros
            for d4 in range(dh // 4):
                vs = []
                for _ in range(4):
                    vs.append(plsc.load_gather(table_v, [d16, k16, l16]))
                    d16 = d16 + 1
                for j, v in enumerate(vs):
                    rows_v[d4 * 4 + j, pl.ds(g * 16, 16)] = v
            return carry

        lax.fori_loop(0, groups, body, 0)
        pltpu.sync_copy(rows_v, out_hbm.at[b, pl.ds(d0, dh)])

    return gather


def kernel(input, embed):
    b, t, c = input.shape
    input_t = jnp.swapaxes(input, 1, 2)
    ind3, loss_sum, table3 = _dist_argmin(input_t, embed)
    qpad = _make_gather(b)(table3, ind3)
    quantize = jnp.swapaxes(qpad[:, :, :t], 1, 2)
    loss = loss_sum * (_BETA / (b * t * c))
    return quantize, loss, ind3[:, 0, :t]
